# baseline (speedup 1.0000x reference)
import jax
import jax.numpy as jnp
from jax import lax
from jax.experimental import pallas as pl
from jax.experimental.pallas import tpu as pltpu
from jax.experimental.pallas import tpu_sc as plsc

VOCAB = 1000000
EMBED_DIM = 64
BATCH = 16384

NUM_CORES = 2
NUM_SUBCORES = 16
LANES = 16

NW = NUM_CORES * NUM_SUBCORES
B_PER_W = BATCH // NW
CHUNK = 128
N_CHUNKS = B_PER_W // CHUNK


def _sc_body(tgt_idx_hbm, ctx_idx_hbm, in2_hbm, out2_hbm, score_hbm,
             tgt_idx_v, ctx_idx_v, tgt_half_v, ctx_half_v,
             tgt_rows_v, ctx_rows_v, score_v, sem_t, sem_c):
    wid = lax.axis_index("s") * NUM_CORES + lax.axis_index("c")
    base = wid * B_PER_W

    pltpu.sync_copy(tgt_idx_hbm.at[pl.ds(base, B_PER_W)], tgt_idx_v)
    pltpu.sync_copy(ctx_idx_hbm.at[pl.ds(base, B_PER_W)], ctx_idx_v)

    def halve(g, c):
        s = pl.ds(g * LANES, LANES)
        tgt_half_v[s] = tgt_idx_v[s] >> 1
        ctx_half_v[s] = ctx_idx_v[s] >> 1
        return c

    lax.fori_loop(0, B_PER_W // LANES, halve, 0)

    lane_iota = lax.iota(jnp.int32, LANES)

    def chunk_body(ck, c):
        row0 = ck * CHUNK
        cp_t = pltpu.async_copy(
            in2_hbm.at[tgt_half_v.at[pl.ds(row0, CHUNK)]], tgt_rows_v, sem_t)
        cp_c = pltpu.async_copy(
            out2_hbm.at[ctx_half_v.at[pl.ds(row0, CHUNK)]], ctx_rows_v, sem_c)
        cp_t.wait()
        cp_c.wait()

        def group(g, c2):
            s = pl.ds(row0 + g * LANES, LANES)
            rows = g * LANES + lane_iota
            tpar = (tgt_idx_v[s] & 1) * EMBED_DIM
            cpar = (ctx_idx_v[s] & 1) * EMBED_DIM
            acc = jnp.zeros((LANES,), jnp.float32)
            for d in range(EMBED_DIM):
                tv = plsc.load_gather(tgt_rows_v, [rows, tpar + d])
                cv = plsc.load_gather(ctx_rows_v, [rows, cpar + d])
                acc = acc + tv * cv
            score_v[s] = acc
            return c2

        lax.fori_loop(0, CHUNK // LANES, group, 0)
        return c

    lax.fori_loop(0, N_CHUNKS, chunk_body, 0)

    pltpu.sync_copy(score_v, score_hbm.at[pl.ds(base, B_PER_W)])


@jax.jit
def _w2v_scores(tgt_idx, ctx_idx, in_emb, out_emb):
    in2 = in_emb.reshape(VOCAB // 2, 2 * EMBED_DIM)
    out2 = out_emb.reshape(VOCAB // 2, 2 * EMBED_DIM)
    mesh = plsc.VectorSubcoreMesh(
        core_axis_name="c", subcore_axis_name="s",
        num_cores=NUM_CORES, num_subcores=NUM_SUBCORES)
    return pl.kernel(
        _sc_body,
        out_type=jax.ShapeDtypeStruct((BATCH,), jnp.float32),
        mesh=mesh,
        scratch_types=[
            pltpu.VMEM((B_PER_W,), jnp.int32),
            pltpu.VMEM((B_PER_W,), jnp.int32),
            pltpu.VMEM((B_PER_W,), jnp.int32),
            pltpu.VMEM((B_PER_W,), jnp.int32),
            pltpu.VMEM((CHUNK, 2 * EMBED_DIM), jnp.float32),
            pltpu.VMEM((CHUNK, 2 * EMBED_DIM), jnp.float32),
            pltpu.VMEM((B_PER_W,), jnp.float32),
            pltpu.SemaphoreType.DMA,
            pltpu.SemaphoreType.DMA,
        ],
        compiler_params=pltpu.CompilerParams(needs_layout_passes=False),
    )(tgt_idx, ctx_idx, in2, out2)


def kernel(target_word_idx, context_word_idx, in_emb, out_emb):
    tgt = target_word_idx.astype(jnp.int32)
    ctx = context_word_idx.astype(jnp.int32)
    return _w2v_scores(tgt, ctx, in_emb, out_emb)
